# Initial kernel scaffold; baseline (speedup 1.0000x reference)
import functools

import jax
import jax.numpy as jnp
from jax import lax
from jax.experimental import pallas as pl
from jax.experimental.pallas import tpu as pltpu
from jax.experimental.pallas import tpu_sc as plsc

NUM_GENES = 2000
EMBED_DIM = 32
BATCH = 1024
NC = 2
NS = 16
NW = NC * NS
ROWS_PER_W = BATCH // NW
PAD_LOOKUPS = 2048


def _norm_body(x_ref, o_ref):
    x = x_ref[...]
    for k in range(4):
        c = x[:, k * EMBED_DIM:(k + 1) * EMBED_DIM]
        s = jnp.sum(c * c, axis=1, keepdims=True)
        denom = jnp.maximum(jnp.sqrt(s), 1e-12)
        o_ref[:, k * EMBED_DIM:(k + 1) * EMBED_DIM] = c / denom


def _normalize_table(embedding_mat):
    emb2d = embedding_mat.reshape(NUM_GENES, 4 * EMBED_DIM)
    out = pl.pallas_call(
        _norm_body,
        out_shape=jax.ShapeDtypeStruct((NUM_GENES, 4 * EMBED_DIM), jnp.float32),
    )(emb2d)
    return out.reshape(NUM_GENES * 4, EMBED_DIM)


_sc_mesh = plsc.VectorSubcoreMesh(
    core_axis_name="c", subcore_axis_name="s", num_cores=NC, num_subcores=NS
)


@functools.partial(
    pl.kernel,
    out_type=jax.ShapeDtypeStruct((BATCH, NUM_GENES, EMBED_DIM), jnp.float32),
    mesh=_sc_mesh,
    scratch_types=[
        pltpu.VMEM((PAD_LOOKUPS,), jnp.int32),
        pltpu.VMEM((PAD_LOOKUPS,), jnp.int32),
        pltpu.VMEM((PAD_LOOKUPS, EMBED_DIM), jnp.float32),
        pltpu.SemaphoreType.DMA,
    ],
)
def _sc_gather(table_hbm, gs_hbm, ramp_hbm, out_hbm, ramp_v, idx_v, rows_v, sem):
    wid = lax.axis_index("s") * NC + lax.axis_index("c")
    pltpu.sync_copy(ramp_hbm, ramp_v)
    zeros = jnp.zeros((16,), jnp.int32)
    for t in range(3):
        idx_v[pl.ds(NUM_GENES + t * 16, 16)] = zeros

    def row_body(t, carry):
        n = wid * ROWS_PER_W + t
        pltpu.sync_copy(gs_hbm.at[n], idx_v.at[pl.ds(0, NUM_GENES)])

        def add_body(j, c):
            o = j * 16
            idx_v[pl.ds(o, 16)] = idx_v[pl.ds(o, 16)] + ramp_v[pl.ds(o, 16)]
            return c

        lax.fori_loop(0, NUM_GENES // 16, add_body, 0)
        copies = [
            pltpu.async_copy(
                table_hbm.at[idx_v.at[pl.ds(g * 128, 128)]],
                rows_v.at[pl.ds(g * 128, 128)],
                sem,
            )
            for g in range(PAD_LOOKUPS // 128)
        ]
        for cp in copies:
            cp.wait()
        pltpu.sync_copy(rows_v.at[pl.ds(0, NUM_GENES)], out_hbm.at[n])
        return carry

    lax.fori_loop(0, ROWS_PER_W, row_body, 0)


def kernel(gene_seq, embedding_mat):
    table = _normalize_table(embedding_mat)
    ramp = jnp.concatenate(
        [
            jnp.arange(NUM_GENES, dtype=jnp.int32) * 4,
            jnp.zeros((PAD_LOOKUPS - NUM_GENES,), jnp.int32),
        ]
    )
    return _sc_gather(table, gene_seq, ramp)


# speedup vs baseline: 8.8190x; 8.8190x over previous
import functools

import jax
import jax.numpy as jnp
from jax import lax
from jax.experimental import pallas as pl
from jax.experimental.pallas import tpu as pltpu
from jax.experimental.pallas import tpu_sc as plsc

NUM_GENES = 2000
EMBED_DIM = 32
BATCH = 1024
NC = 2
NS = 16
NW = NC * NS
ROWS_PER_W = BATCH // NW
PAD_LOOKUPS = 2048


def _norm_body(x_ref, o_ref):
    x = x_ref[...]
    for k in range(4):
        c = x[:, k * EMBED_DIM:(k + 1) * EMBED_DIM]
        s = jnp.sum(c * c, axis=1, keepdims=True)
        denom = jnp.maximum(jnp.sqrt(s), 1e-12)
        o_ref[:, k * EMBED_DIM:(k + 1) * EMBED_DIM] = c / denom


def _normalize_table(embedding_mat):
    emb2d = embedding_mat.reshape(NUM_GENES, 4 * EMBED_DIM)
    out = pl.pallas_call(
        _norm_body,
        out_shape=jax.ShapeDtypeStruct((NUM_GENES, 4 * EMBED_DIM), jnp.float32),
    )(emb2d)
    return out.reshape(NUM_GENES * 4, EMBED_DIM)


_sc_mesh = plsc.VectorSubcoreMesh(
    core_axis_name="c", subcore_axis_name="s", num_cores=NC, num_subcores=NS
)


@functools.partial(
    pl.kernel,
    out_type=jax.ShapeDtypeStruct((BATCH, NUM_GENES, EMBED_DIM), jnp.float32),
    mesh=_sc_mesh,
    scratch_types=[
        pltpu.VMEM((PAD_LOOKUPS,), jnp.int32),
        pltpu.VMEM((PAD_LOOKUPS,), jnp.int32),
        pltpu.VMEM((PAD_LOOKUPS, EMBED_DIM), jnp.float32),
        pltpu.SemaphoreType.DMA,
    ],
    compiler_params=pltpu.CompilerParams(use_tc_tiling_on_sc=False),
)
def _sc_gather(table_hbm, gs_hbm, ramp_hbm, out_hbm, ramp_v, idx_v, rows_v, sem):
    wid = lax.axis_index("s") * NC + lax.axis_index("c")
    pltpu.sync_copy(ramp_hbm, ramp_v)
    zeros = jnp.zeros((16,), jnp.int32)
    for t in range(3):
        idx_v[pl.ds(NUM_GENES + t * 16, 16)] = zeros

    def row_body(t, carry):
        n = wid * ROWS_PER_W + t
        pltpu.sync_copy(gs_hbm.at[n], idx_v.at[pl.ds(0, NUM_GENES)])

        def add_body(j, c):
            o = j * 16
            idx_v[pl.ds(o, 16)] = idx_v[pl.ds(o, 16)] + ramp_v[pl.ds(o, 16)]
            return c

        lax.fori_loop(0, NUM_GENES // 16, add_body, 0)
        copies = [
            pltpu.async_copy(
                table_hbm.at[idx_v.at[pl.ds(g * 128, 128)]],
                rows_v.at[pl.ds(g * 128, 128)],
                sem,
            )
            for g in range(PAD_LOOKUPS // 128)
        ]
        for cp in copies:
            cp.wait()
        pltpu.sync_copy(rows_v.at[pl.ds(0, NUM_GENES)], out_hbm.at[n])
        return carry

    lax.fori_loop(0, ROWS_PER_W, row_body, 0)


def kernel(gene_seq, embedding_mat):
    table = _normalize_table(embedding_mat)
    ramp = jnp.concatenate(
        [
            jnp.arange(NUM_GENES, dtype=jnp.int32) * 4,
            jnp.zeros((PAD_LOOKUPS - NUM_GENES,), jnp.int32),
        ]
    )
    return _sc_gather(table, gene_seq, ramp)


# speedup vs baseline: 8.8614x; 1.0048x over previous
import functools

import jax
import jax.numpy as jnp
from jax import lax
from jax.experimental import pallas as pl
from jax.experimental.pallas import tpu as pltpu
from jax.experimental.pallas import tpu_sc as plsc

NUM_GENES = 2000
EMBED_DIM = 32
BATCH = 1024
NC = 2
NS = 16
NW = NC * NS
ROWS_PER_W = BATCH // NW
PAD_LOOKUPS = 2048


def _norm_body(x_ref, o_ref):
    x = x_ref[...]
    for k in range(4):
        c = x[:, k * EMBED_DIM:(k + 1) * EMBED_DIM]
        s = jnp.sum(c * c, axis=1, keepdims=True)
        denom = jnp.maximum(jnp.sqrt(s), 1e-12)
        o_ref[:, k * EMBED_DIM:(k + 1) * EMBED_DIM] = c / denom


def _normalize_table(embedding_mat):
    emb2d = embedding_mat.reshape(NUM_GENES, 4 * EMBED_DIM)
    out = pl.pallas_call(
        _norm_body,
        out_shape=jax.ShapeDtypeStruct((NUM_GENES, 4 * EMBED_DIM), jnp.float32),
    )(emb2d)
    return out.reshape(NUM_GENES * 4, EMBED_DIM)


_sc_mesh = plsc.VectorSubcoreMesh(
    core_axis_name="c", subcore_axis_name="s", num_cores=NC, num_subcores=NS
)

CHUNK = 1024
_CLEN = (CHUNK, NUM_GENES - CHUNK)
_NGATH = CHUNK // 128


@functools.partial(
    pl.kernel,
    out_type=jax.ShapeDtypeStruct((BATCH, NUM_GENES, EMBED_DIM), jnp.float32),
    mesh=_sc_mesh,
    scratch_types=[
        pltpu.VMEM((PAD_LOOKUPS,), jnp.int32),
        pltpu.VMEM((CHUNK,), jnp.int32),
        pltpu.VMEM((CHUNK,), jnp.int32),
        pltpu.VMEM((CHUNK, EMBED_DIM), jnp.float32),
        pltpu.VMEM((CHUNK, EMBED_DIM), jnp.float32),
        pltpu.SemaphoreType.DMA,
        pltpu.SemaphoreType.DMA,
        pltpu.SemaphoreType.DMA,
        pltpu.SemaphoreType.DMA,
    ],
    compiler_params=pltpu.CompilerParams(use_tc_tiling_on_sc=False),
)
def _sc_gather(table_hbm, gs_hbm, ramp_hbm, out_hbm,
               ramp_v, idx0, idx1, rows0, rows1, gsem0, gsem1, wsem0, wsem1):
    wid = lax.axis_index("s") * NC + lax.axis_index("c")
    idx_b = (idx0, idx1)
    rows_b = (rows0, rows1)
    gsem_b = (gsem0, gsem1)
    wsem_b = (wsem0, wsem1)
    pltpu.sync_copy(ramp_hbm, ramp_v)
    zeros = jnp.zeros((16,), jnp.int32)
    for t in range(3):
        idx1[pl.ds(_CLEN[1] + t * 16, 16)] = zeros

    def _wait_write(c):
        pltpu.make_async_copy(
            rows_b[c].at[pl.ds(0, _CLEN[c])],
            out_hbm.at[0, pl.ds(0, _CLEN[c])],
            wsem_b[c],
        ).wait()

    def row_body(t, carry):
        n = wid * ROWS_PER_W + t
        gathers = []
        for c in range(2):
            L = _CLEN[c]

            @pl.when(t > 0)
            def _():
                _wait_write(c)

            pltpu.sync_copy(gs_hbm.at[n, pl.ds(c * CHUNK, L)],
                            idx_b[c].at[pl.ds(0, L)])

            def add_body(j, cc, c=c):
                o = j * 16
                idx_b[c][pl.ds(o, 16)] = (
                    idx_b[c][pl.ds(o, 16)] + ramp_v[pl.ds(c * CHUNK + o, 16)]
                )
                return cc

            lax.fori_loop(0, L // 16, add_body, 0)
            gathers.append([
                pltpu.async_copy(
                    table_hbm.at[idx_b[c].at[pl.ds(g * 128, 128)]],
                    rows_b[c].at[pl.ds(g * 128, 128)],
                    gsem_b[c],
                )
                for g in range(_NGATH)
            ])
        for c in range(2):
            for cp in gathers[c]:
                cp.wait()
            pltpu.async_copy(
                rows_b[c].at[pl.ds(0, _CLEN[c])],
                out_hbm.at[n, pl.ds(c * CHUNK, _CLEN[c])],
                wsem_b[c],
            )
        return carry

    lax.fori_loop(0, ROWS_PER_W, row_body, 0)
    for c in range(2):
        _wait_write(c)


def kernel(gene_seq, embedding_mat):
    table = _normalize_table(embedding_mat)
    ramp = jnp.concatenate(
        [
            jnp.arange(NUM_GENES, dtype=jnp.int32) * 4,
            jnp.zeros((PAD_LOOKUPS - NUM_GENES,), jnp.int32),
        ]
    )
    return _sc_gather(table, gene_seq, ramp)


# speedup vs baseline: 10.9127x; 1.2315x over previous
import functools

import jax
import jax.numpy as jnp
from jax import lax
from jax.experimental import pallas as pl
from jax.experimental.pallas import tpu as pltpu
from jax.experimental.pallas import tpu_sc as plsc

NUM_GENES = 2000
EMBED_DIM = 32
BATCH = 1024
NC = 2
NS = 16
NW = NC * NS
GROUP = 8
NGROUPS = NUM_GENES // GROUP
GROUPS_PER_W = -(-NGROUPS // NW)


def _norm_body(x_ref, o_ref):
    x = x_ref[...]
    for k in range(4):
        c = x[:, k * EMBED_DIM:(k + 1) * EMBED_DIM]
        s = jnp.sum(c * c, axis=1, keepdims=True)
        denom = jnp.maximum(jnp.sqrt(s), 1e-12)
        o_ref[:, k * EMBED_DIM:(k + 1) * EMBED_DIM] = c / denom


def _normalize_table(embedding_mat):
    emb2d = embedding_mat.reshape(NUM_GENES, 4 * EMBED_DIM)
    return pl.pallas_call(
        _norm_body,
        out_shape=jax.ShapeDtypeStruct((NUM_GENES, 4 * EMBED_DIM), jnp.float32),
    )(emb2d)


_sc_mesh = plsc.VectorSubcoreMesh(
    core_axis_name="c", subcore_axis_name="s", num_cores=NC, num_subcores=NS
)


@functools.partial(
    pl.kernel,
    out_type=jax.ShapeDtypeStruct((NUM_GENES, EMBED_DIM, BATCH), jnp.float32),
    mesh=_sc_mesh,
    scratch_types=[
        pltpu.VMEM((GROUP * BATCH,), jnp.int32),
        pltpu.VMEM((GROUP * 4 * EMBED_DIM,), jnp.float32),
        pltpu.VMEM((EMBED_DIM, BATCH), jnp.float32),
        pltpu.VMEM((EMBED_DIM, BATCH), jnp.float32),
        pltpu.SemaphoreType.DMA,
        pltpu.SemaphoreType.DMA,
    ],
    compiler_params=pltpu.CompilerParams(
        use_tc_tiling_on_sc=True, needs_layout_passes=False
    ),
)
def _sc_select(table_hbm, gst_hbm, out_hbm, kv, slab, ob0, ob1, ws0, ws1):
    wid = lax.axis_index("s") * NC + lax.axis_index("c")
    ob = (ob0, ob1)
    ws = (ws0, ws1)

    def group_body(t, carry):
        g = t * NW + wid

        @pl.when(g < NGROUPS)
        def _():
            m0 = g * GROUP
            pltpu.sync_copy(gst_hbm.at[pl.ds(m0 * BATCH, GROUP * BATCH)], kv)
            pltpu.sync_copy(
                table_hbm.at[pl.ds(m0 * 4 * EMBED_DIM, GROUP * 4 * EMBED_DIM)],
                slab,
            )
            for r in range(GROUP):
                buf = ob[r % 2]

                def vec_body(j, cc, r=r, buf=buf):
                    kvec = kv[pl.ds(r * BATCH + j * 16, 16)]
                    base = kvec * EMBED_DIM + (r * 4 * EMBED_DIM)
                    for d in range(EMBED_DIM):
                        val = plsc.load_gather(slab, [base + d])
                        buf[d, pl.ds(j * 16, 16)] = val
                    return cc

                if r >= 2:
                    pltpu.make_async_copy(buf, out_hbm.at[0], ws[r % 2]).wait()
                else:
                    @pl.when(t > 0)
                    def _(buf=buf, r=r):
                        pltpu.make_async_copy(
                            buf, out_hbm.at[0], ws[r % 2]
                        ).wait()

                lax.fori_loop(0, BATCH // 16, vec_body, 0)
                pltpu.async_copy(buf, out_hbm.at[m0 + r], ws[r % 2])
        return carry

    lax.fori_loop(0, GROUPS_PER_W, group_body, 0)
    for r in range(2):
        pltpu.make_async_copy(ob[r], out_hbm.at[0], ws[r]).wait()


def kernel(gene_seq, embedding_mat):
    table = _normalize_table(embedding_mat).reshape(-1)
    gst = jnp.transpose(gene_seq).reshape(-1)
    out = _sc_select(table, gst)
    return jnp.transpose(out, (2, 0, 1))


# speedup vs baseline: 17.4088x; 1.5953x over previous
import functools

import jax
import jax.numpy as jnp
from jax import lax
from jax.experimental import pallas as pl
from jax.experimental.pallas import tpu as pltpu
from jax.experimental.pallas import tpu_sc as plsc

NUM_GENES = 2000
EMBED_DIM = 32
BATCH = 1024
NC = 2
NS = 16
NW = NC * NS
GROUP = 8
NGROUPS = NUM_GENES // GROUP
GROUPS_PER_W = -(-NGROUPS // NW)


def _norm_body(x_ref, o_ref):
    x = x_ref[...]
    for k in range(4):
        c = x[:, k * EMBED_DIM:(k + 1) * EMBED_DIM]
        s = jnp.sum(c * c, axis=1, keepdims=True)
        denom = jnp.maximum(jnp.sqrt(s), 1e-12)
        o_ref[:, k * EMBED_DIM:(k + 1) * EMBED_DIM] = c / denom


def _normalize_table(embedding_mat):
    emb2d = embedding_mat.reshape(NUM_GENES, 4 * EMBED_DIM)
    return pl.pallas_call(
        _norm_body,
        out_shape=jax.ShapeDtypeStruct((NUM_GENES, 4 * EMBED_DIM), jnp.float32),
    )(emb2d)


_sc_mesh = plsc.VectorSubcoreMesh(
    core_axis_name="c", subcore_axis_name="s", num_cores=NC, num_subcores=NS
)


@functools.partial(
    pl.kernel,
    out_type=jax.ShapeDtypeStruct((NUM_GENES, EMBED_DIM, BATCH), jnp.float32),
    mesh=_sc_mesh,
    scratch_types=[
        pltpu.VMEM((GROUP * BATCH,), jnp.int32),
        pltpu.VMEM((GROUP * 4 * EMBED_DIM,), jnp.float32),
        pltpu.VMEM((EMBED_DIM, BATCH), jnp.float32),
        pltpu.VMEM((EMBED_DIM, BATCH), jnp.float32),
        pltpu.SemaphoreType.DMA,
        pltpu.SemaphoreType.DMA,
    ],
    compiler_params=pltpu.CompilerParams(
        use_tc_tiling_on_sc=True, needs_layout_passes=False
    ),
)
def _sc_select(table_hbm, gst_hbm, out_hbm, kv, slab, ob0, ob1, ws0, ws1):
    wid = lax.axis_index("s") * NC + lax.axis_index("c")
    ob = (ob0, ob1)
    ws = (ws0, ws1)

    def group_body(t, carry):
        g = t * NW + wid

        @pl.when(g < NGROUPS)
        def _():
            m0 = g * GROUP
            pltpu.sync_copy(gst_hbm.at[pl.ds(m0 * BATCH, GROUP * BATCH)], kv)
            pltpu.sync_copy(
                table_hbm.at[pl.ds(m0 * 4 * EMBED_DIM, GROUP * 4 * EMBED_DIM)],
                slab,
            )
            for r in range(GROUP):
                buf = ob[r % 2]

                if r >= 2:
                    pltpu.make_async_copy(buf, out_hbm.at[0], ws[r % 2]).wait()
                else:
                    @pl.when(t > 0)
                    def _(buf=buf, r=r):
                        pltpu.make_async_copy(
                            buf, out_hbm.at[0], ws[r % 2]
                        ).wait()

                @plsc.parallel_loop(0, BATCH // 16, unroll=4)
                def vec_body(j, r=r, buf=buf):
                    kvec = kv[pl.ds(r * BATCH + j * 16, 16)]
                    base = kvec * EMBED_DIM + (r * 4 * EMBED_DIM)
                    for d in range(EMBED_DIM):
                        val = plsc.load_gather(slab, [base + d])
                        buf[d, pl.ds(j * 16, 16)] = val

                pltpu.async_copy(buf, out_hbm.at[m0 + r], ws[r % 2])
        return carry

    lax.fori_loop(0, GROUPS_PER_W, group_body, 0)
    for r in range(2):
        pltpu.make_async_copy(ob[r], out_hbm.at[0], ws[r]).wait()


def kernel(gene_seq, embedding_mat):
    table = _normalize_table(embedding_mat).reshape(-1)
    gst = jnp.transpose(gene_seq).reshape(-1)
    out = _sc_select(table, gst)
    return jnp.transpose(out, (2, 0, 1))


# speedup vs baseline: 17.5080x; 1.0057x over previous
import functools

import jax
import jax.numpy as jnp
from jax import lax
from jax.experimental import pallas as pl
from jax.experimental.pallas import tpu as pltpu
from jax.experimental.pallas import tpu_sc as plsc

NUM_GENES = 2000
EMBED_DIM = 32
BATCH = 1024
NC = 2
NS = 16
NW = NC * NS
GROUP = 8
NGROUPS = NUM_GENES // GROUP
GROUPS_PER_W = -(-NGROUPS // NW)


def _norm_body(x_ref, o_ref):
    x = x_ref[...]
    for k in range(4):
        c = x[:, k * EMBED_DIM:(k + 1) * EMBED_DIM]
        s = jnp.sum(c * c, axis=1, keepdims=True)
        denom = jnp.maximum(jnp.sqrt(s), 1e-12)
        o_ref[:, k * EMBED_DIM:(k + 1) * EMBED_DIM] = c / denom


def _normalize_table(embedding_mat):
    emb2d = embedding_mat.reshape(NUM_GENES, 4 * EMBED_DIM)
    return pl.pallas_call(
        _norm_body,
        out_shape=jax.ShapeDtypeStruct((NUM_GENES, 4 * EMBED_DIM), jnp.float32),
    )(emb2d)


_sc_mesh = plsc.VectorSubcoreMesh(
    core_axis_name="c", subcore_axis_name="s", num_cores=NC, num_subcores=NS
)


@functools.partial(
    pl.kernel,
    out_type=jax.ShapeDtypeStruct((NUM_GENES, EMBED_DIM, BATCH), jnp.float32),
    mesh=_sc_mesh,
    scratch_types=[
        pltpu.VMEM((GROUP * BATCH,), jnp.int32),
        pltpu.VMEM((GROUP * 4 * EMBED_DIM,), jnp.float32),
        pltpu.VMEM((EMBED_DIM, BATCH), jnp.float32),
        pltpu.VMEM((EMBED_DIM, BATCH), jnp.float32),
        pltpu.VMEM((EMBED_DIM, BATCH), jnp.float32),
        pltpu.SemaphoreType.DMA,
        pltpu.SemaphoreType.DMA,
        pltpu.SemaphoreType.DMA,
    ],
    compiler_params=pltpu.CompilerParams(
        use_tc_tiling_on_sc=True, needs_layout_passes=False
    ),
)
def _sc_select(table_hbm, gst_hbm, out_hbm, kv, slab, ob0, ob1, ob2,
               ws0, ws1, ws2):
    wid = lax.axis_index("s") * NC + lax.axis_index("c")
    NBUF = 3
    NSPLIT = 4
    SUB = EMBED_DIM // NSPLIT
    ob = (ob0, ob1, ob2)
    ws = (ws0, ws1, ws2)

    def _wait_write(buf, sem):
        for p in range(NSPLIT):
            pltpu.make_async_copy(
                buf.at[pl.ds(p * SUB, SUB)],
                out_hbm.at[0, pl.ds(p * SUB, SUB)],
                sem,
            ).wait()

    def group_body(t, carry):
        g = t * NW + wid

        @pl.when(g < NGROUPS)
        def _():
            m0 = g * GROUP
            pltpu.sync_copy(gst_hbm.at[pl.ds(m0 * BATCH, GROUP * BATCH)], kv)
            pltpu.sync_copy(
                table_hbm.at[pl.ds(m0 * 4 * EMBED_DIM, GROUP * 4 * EMBED_DIM)],
                slab,
            )
            for r in range(GROUP):
                buf = ob[r % NBUF]
                sem = ws[r % NBUF]

                if r >= NBUF:
                    _wait_write(buf, sem)
                else:
                    @pl.when(t > 0)
                    def _(buf=buf, sem=sem):
                        _wait_write(buf, sem)

                @plsc.parallel_loop(0, BATCH // 16, unroll=4)
                def vec_body(j, r=r, buf=buf):
                    kvec = kv[pl.ds(r * BATCH + j * 16, 16)]
                    base = kvec * EMBED_DIM + (r * 4 * EMBED_DIM)
                    for d in range(EMBED_DIM):
                        val = plsc.load_gather(slab, [base + d])
                        buf[d, pl.ds(j * 16, 16)] = val

                for p in range(NSPLIT):
                    pltpu.async_copy(
                        buf.at[pl.ds(p * SUB, SUB)],
                        out_hbm.at[m0 + r, pl.ds(p * SUB, SUB)],
                        sem,
                    )
        return carry

    lax.fori_loop(0, GROUPS_PER_W, group_body, 0)
    for r in range(NBUF):
        _wait_write(ob[r], ws[r])


def kernel(gene_seq, embedding_mat):
    table = _normalize_table(embedding_mat).reshape(-1)
    gst = jnp.transpose(gene_seq).reshape(-1)
    out = _sc_select(table, gst)
    return jnp.transpose(out, (2, 0, 1))
